# baseline (speedup 1.0000x reference)
import functools

import jax
import jax.numpy as jnp
from jax import lax
from jax.experimental import pallas as pl
from jax.experimental.pallas import tpu as pltpu
from jax.experimental.pallas import tpu_sc as plsc

NUM_EMB = 1_000_000
DIM = 64
BATCH = 4096
HIST = 50

NC = 2
NS = 16
NW = NC * NS
BPW = BATCH // NW
NBUF = 6
AHEAD = 5


def _make_kernel():
    mesh = plsc.VectorSubcoreMesh(core_axis_name="c", subcore_axis_name="s")

    @functools.partial(
        pl.kernel,
        mesh=mesh,
        out_type=jax.ShapeDtypeStruct((BATCH, HIST, DIM), jnp.float32),
        compiler_params=pltpu.CompilerParams(use_tc_tiling_on_sc=True),
        scratch_types=[
            pltpu.VMEM((HIST, BPW), jnp.int32),
            pltpu.VMEM((NBUF, BPW, DIM), jnp.float32),
            pltpu.SemaphoreType.DMA((NBUF,)),
            pltpu.SemaphoreType.DMA((NBUF,)),
        ],
    )
    def emb(table_hbm, idxt_hbm, out_hbm, idx_v, rows_v, gsem, osem):
        wid = lax.axis_index("s") * NC + lax.axis_index("c")
        b0 = wid * BPW
        pltpu.sync_copy(idxt_hbm.at[:, pl.ds(b0, BPW)], idx_v)

        def fire_gathers(h, b):
            row = idx_v.at[h]
            for v0 in range(0, BPW, 16):
                vec = row[pl.ds(v0, 16)]
                for j in range(16):
                    ridx = vec[j]
                    pltpu.make_async_copy(
                        table_hbm.at[pl.ds(ridx, 1)],
                        rows_v.at[b, pl.ds(v0 + j, 1)],
                        gsem.at[b]).start()

        def drain_gathers(b):
            pltpu.make_async_copy(
                out_hbm.at[pl.ds(b0, BPW), 0], rows_v.at[b],
                gsem.at[b]).wait()

        def fire_write(h, b):
            pltpu.make_async_copy(
                rows_v.at[b], out_hbm.at[pl.ds(b0, BPW), h],
                osem.at[b]).start()

        def drain_write(b):
            pltpu.make_async_copy(
                rows_v.at[b], out_hbm.at[pl.ds(b0, BPW), 0],
                osem.at[b]).wait()

        for h in range(AHEAD):
            fire_gathers(h, h)

        def step(h, carry):
            b = lax.rem(h, NBUF)
            drain_gathers(b)
            fire_write(h, b)
            b2 = lax.rem(h + AHEAD, NBUF)

            @pl.when(h + AHEAD <= HIST - 1)
            def _():
                @pl.when(h >= 1)
                def _():
                    drain_write(b2)
                fire_gathers(h + AHEAD, b2)

            return carry

        lax.fori_loop(0, HIST, step, 0, unroll=False)
        for b in range(NBUF):
            drain_write(b)

    return emb


_emb = _make_kernel()


def kernel(x, weight):
    return _emb(weight, x.T)


# speedup vs baseline: 1.3169x; 1.3169x over previous
import functools

import jax
import jax.numpy as jnp
from jax import lax
from jax.experimental import pallas as pl
from jax.experimental.pallas import tpu as pltpu
from jax.experimental.pallas import tpu_sc as plsc

NUM_EMB = 1_000_000
DIM = 64
BATCH = 4096
HIST = 50

NC = 2
NS = 16
NW = NC * NS
BPW = BATCH // NW
NBUF = 4
AHEAD = 3


def _make_kernel():
    mesh = plsc.VectorSubcoreMesh(core_axis_name="c", subcore_axis_name="s")

    @functools.partial(
        pl.kernel,
        mesh=mesh,
        out_type=jax.ShapeDtypeStruct((BATCH, HIST, DIM), jnp.float32),
        compiler_params=pltpu.CompilerParams(use_tc_tiling_on_sc=True),
        scratch_types=[
            pltpu.VMEM((HIST, BPW), jnp.int32),
            pltpu.VMEM((NBUF, BPW, DIM), jnp.float32),
            pltpu.SemaphoreType.DMA((NBUF,)),
            pltpu.SemaphoreType.DMA((NBUF,)),
        ],
    )
    def emb(table3_hbm, idxt_hbm, out_hbm, idx_v, rows_v, gsem, osem):
        table_hbm = table3_hbm.at[0]
        wid = lax.axis_index("s") * NC + lax.axis_index("c")
        b0 = wid * BPW
        pltpu.sync_copy(idxt_hbm.at[:, pl.ds(b0, BPW)], idx_v)

        def fire_gathers(h, b):
            row = idx_v.at[h]
            for v0 in range(0, BPW, 16):
                vec = row[pl.ds(v0, 16)]
                for j in range(16):
                    ridx = vec[j]
                    pltpu.make_async_copy(
                        table_hbm.at[pl.ds(ridx, 1)],
                        rows_v.at[b, pl.ds(v0 + j, 1)],
                        gsem.at[b]).start()

        def drain_gathers(b):
            pltpu.make_async_copy(
                out_hbm.at[pl.ds(b0, BPW), 0], rows_v.at[b],
                gsem.at[b]).wait()

        def fire_write(h, b):
            pltpu.make_async_copy(
                rows_v.at[b], out_hbm.at[pl.ds(b0, BPW), h],
                osem.at[b]).start()

        def drain_write(b):
            pltpu.make_async_copy(
                rows_v.at[b], out_hbm.at[pl.ds(b0, BPW), 0],
                osem.at[b]).wait()

        for h in range(AHEAD):
            fire_gathers(h, h)

        def step(h, carry):
            b = lax.rem(h, NBUF)
            drain_gathers(b)
            fire_write(h, b)
            b2 = lax.rem(h + AHEAD, NBUF)

            @pl.when(h + AHEAD <= HIST - 1)
            def _():
                @pl.when(h >= 1)
                def _():
                    drain_write(b2)
                fire_gathers(h + AHEAD, b2)

            return carry

        lax.fori_loop(0, HIST, step, 0, unroll=False)
        for b in range(NBUF):
            drain_write(b)

    return emb


_emb = _make_kernel()


def kernel(x, weight):
    return _emb(weight[None], x.T)


# speedup vs baseline: 1.4303x; 1.0862x over previous
import functools

import jax
import jax.numpy as jnp
from jax import lax
from jax.experimental import pallas as pl
from jax.experimental.pallas import tpu as pltpu
from jax.experimental.pallas import tpu_sc as plsc

NUM_EMB = 1_000_000
DIM = 64
BATCH = 4096
HIST = 50

NC = 2
NS = 16
NW = NC * NS
BPW = BATCH // NW
NBUF = 4
AHEAD = 3


def _make_kernel():
    mesh = plsc.VectorSubcoreMesh(core_axis_name="c", subcore_axis_name="s")

    @functools.partial(
        pl.kernel,
        mesh=mesh,
        out_type=jax.ShapeDtypeStruct((1, BATCH, HIST, DIM), jnp.float32),
        compiler_params=pltpu.CompilerParams(use_tc_tiling_on_sc=True),
        scratch_types=[
            pltpu.VMEM((HIST, BPW), jnp.int32),
            pltpu.VMEM((NBUF, BPW, DIM), jnp.float32),
            pltpu.SemaphoreType.DMA((NBUF,)),
            pltpu.SemaphoreType.DMA((NBUF,)),
        ],
    )
    def emb(table3_hbm, idxt_hbm, out4_hbm, idx_v, rows_v, gsem, osem):
        table_hbm = table3_hbm.at[0]
        out_hbm = out4_hbm.at[0]
        wid = lax.axis_index("s") * NC + lax.axis_index("c")
        b0 = wid * BPW
        pltpu.sync_copy(idxt_hbm.at[:, pl.ds(b0, BPW)], idx_v)

        def fire_gathers(h, b):
            row = idx_v.at[h]
            for v0 in range(0, BPW, 16):
                vec = row[pl.ds(v0, 16)]
                for j in range(16):
                    ridx = vec[j]
                    pltpu.make_async_copy(
                        table_hbm.at[pl.ds(ridx, 1)],
                        rows_v.at[b, pl.ds(v0 + j, 1)],
                        gsem.at[b]).start()

        def drain_gathers(b):
            pltpu.make_async_copy(
                out_hbm.at[pl.ds(b0, BPW), 0], rows_v.at[b],
                gsem.at[b]).wait()

        def fire_write(h, b):
            pltpu.make_async_copy(
                rows_v.at[b], out_hbm.at[pl.ds(b0, BPW), h],
                osem.at[b]).start()

        def drain_write(b):
            pltpu.make_async_copy(
                rows_v.at[b], out_hbm.at[pl.ds(b0, BPW), 0],
                osem.at[b]).wait()

        for h in range(AHEAD):
            fire_gathers(h, h)

        def step(h, carry):
            b = lax.rem(h, NBUF)
            drain_gathers(b)
            fire_write(h, b)
            b2 = lax.rem(h + AHEAD, NBUF)

            @pl.when(h + AHEAD <= HIST - 1)
            def _():
                @pl.when(h >= 1)
                def _():
                    drain_write(b2)
                fire_gathers(h + AHEAD, b2)

            return carry

        lax.fori_loop(0, HIST, step, 0, unroll=False)
        for b in range(NBUF):
            drain_write(b)

    return emb


_emb = _make_kernel()


def kernel(x, weight):
    return _emb(weight[None], x.T)[0]
